# baseline (speedup 1.0000x reference)
import functools

import jax
import jax.numpy as jnp
from jax import lax
from jax.experimental import pallas as pl
from jax.experimental.pallas import tpu as pltpu
from jax.experimental.pallas import tpu_sc as plsc

_NUM_CLASSES = 10000
_D = 128
_N = 320000
_NC, _NS, _L = 2, 16, 16
_NW = _NC * _NS
_ROWS_PER_W = _N // _NW
_C = 40
_NCHUNK = _ROWS_PER_W // _C
_NBUF = 5
_MAIN_T = _NCHUNK // _NBUF - 1
_GLAG = 2
_JREGS = _D // _L


def _neg_body(t_ref, o_ref):
    o_ref[...] = -t_ref[...]


_neg_table = pl.pallas_call(
    _neg_body,
    out_shape=jax.ShapeDtypeStruct((_NUM_CLASSES, _D), jnp.float32),
)


@functools.partial(
    pl.kernel,
    out_type=jax.ShapeDtypeStruct((_NW, _L), jnp.float32),
    mesh=plsc.VectorSubcoreMesh(
        core_axis_name="c", subcore_axis_name="s",
        num_cores=_NC, num_subcores=_NS),
    scratch_types=[
        pltpu.VMEM((_ROWS_PER_W,), jnp.int32),
        pltpu.VMEM((_NBUF, _C, _D), jnp.float32),
        pltpu.VMEM((_L,), jnp.float32),
    ] + [pltpu.SemaphoreType.DMA] * (2 * _NBUF),
)
def _sc_sqdist(x_hbm, lbl_hbm, ntbl_hbm, out_hbm,
               idx_all, x_v, acc_v, *sems):
    sem_x = sems[:_NBUF]
    sem_m = sems[_NBUF:]
    wid = lax.axis_index("s") * _NC + lax.axis_index("c")
    base = wid * _ROWS_PER_W

    pltpu.sync_copy(lbl_hbm.at[pl.ds(base, _ROWS_PER_W)], idx_all)

    def start_x(ci, b):
        row0 = base + ci * _C
        pltpu.async_copy(x_hbm.at[pl.ds(row0, _C)], x_v.at[b], sem_x[b])

    def wait_x(ci, b):
        row0 = base + ci * _C
        pltpu.make_async_copy(x_hbm.at[pl.ds(row0, _C)],
                              x_v.at[b], sem_x[b]).wait()

    def start_madd(ci, b):
        pltpu.async_copy(ntbl_hbm.at[idx_all.at[pl.ds(ci * _C, _C)]],
                         x_v.at[b], sem_m[b], add=True)

    def wait_madd(b):
        pltpu.make_async_copy(ntbl_hbm.at[pl.ds(0, _C)],
                              x_v.at[b], sem_m[b]).wait()

    def compute(b, accs):
        xb = x_v.at[b]

        def row_body(r2, a):
            new = list(a)
            for h in range(2):
                for j in range(_JREGS):
                    dv = xb[r2 * 2 + h, pl.ds(j * _L, _L)]
                    new[j] = new[j] + dv * dv
            return tuple(new)

        return lax.fori_loop(0, _C // 2, row_body, accs)

    for b in range(_NBUF):
        start_x(b, b)
    for ci in range(_GLAG):
        wait_x(ci, ci)
        start_madd(ci, ci)

    def ring_turn(t, accs):
        for b in range(_NBUF):
            ci = t * _NBUF + b
            wait_madd(b)
            accs = compute(b, accs)
            start_x(ci + _NBUF, b)
            bg = (b + _GLAG) % _NBUF
            wait_x(ci + _GLAG, bg)
            start_madd(ci + _GLAG, bg)
        return accs

    zero = jnp.zeros((_L,), jnp.float32)
    accs = lax.fori_loop(0, _MAIN_T, ring_turn, (zero,) * _JREGS)

    for k in range(_NBUF):
        ci = _MAIN_T * _NBUF + k
        b = ci % _NBUF
        wait_madd(b)
        accs = compute(b, accs)
        if ci + _GLAG < _NCHUNK:
            bg = (ci + _GLAG) % _NBUF
            wait_x(ci + _GLAG, bg)
            start_madd(ci + _GLAG, bg)

    total = accs[0]
    for j in range(1, _JREGS):
        total = total + accs[j]
    acc_v[...] = total
    pltpu.sync_copy(acc_v, out_hbm.at[wid])


def kernel(inputs, labels, running_mean):
    neg_table = _neg_table(running_mean)
    partials = _sc_sqdist(inputs, labels.astype(jnp.int32), neg_table)
    regularization = jnp.sqrt(jnp.sum(partials))
    return inputs, regularization


# speedup vs baseline: 1.0293x; 1.0293x over previous
import functools

import jax
import jax.numpy as jnp
from jax import lax
from jax.experimental import pallas as pl
from jax.experimental.pallas import tpu as pltpu
from jax.experimental.pallas import tpu_sc as plsc

_NUM_CLASSES = 10000
_D = 128
_N = 320000
_NC, _NS, _L = 2, 16, 16
_NW = _NC * _NS
_ROWS_PER_W = _N // _NW
_C = 80
_NCHUNK = _ROWS_PER_W // _C
_NBUF = 5
_MAIN_T = _NCHUNK // _NBUF - 1
_GLAG = 2
_JREGS = _D // _L
_TROWS = 640
_NEGC = 80


@functools.partial(
    pl.kernel,
    out_type=(
        jax.ShapeDtypeStruct((_NW, _L), jnp.float32),
        jax.ShapeDtypeStruct((_NC * _NUM_CLASSES, _D), jnp.float32),
    ),
    mesh=plsc.VectorSubcoreMesh(
        core_axis_name="c", subcore_axis_name="s",
        num_cores=_NC, num_subcores=_NS),
    scratch_types=[
        pltpu.VMEM((_ROWS_PER_W,), jnp.int32),
        pltpu.VMEM((_NBUF, _C, _D), jnp.float32),
        pltpu.VMEM((_NEGC, _D), jnp.float32),
        pltpu.VMEM((_L,), jnp.float32),
    ] + [pltpu.SemaphoreType.DMA] * (2 * _NBUF),
)
def _sc_sqdist(x_hbm, lbl_hbm, tbl_hbm, out_hbm, ntbl_hbm,
               idx_all, x_v, neg_v, acc_v, *sems):
    sem_x = sems[:_NBUF]
    sem_m = sems[_NBUF:]
    cid = lax.axis_index("c")
    sid = lax.axis_index("s")
    wid = sid * _NC + cid
    base = wid * _ROWS_PER_W

    def start_x(ci, b):
        row0 = base + ci * _C
        pltpu.async_copy(x_hbm.at[pl.ds(row0, _C)], x_v.at[b], sem_x[b])

    def wait_x(ci, b):
        row0 = base + ci * _C
        pltpu.make_async_copy(x_hbm.at[pl.ds(row0, _C)],
                              x_v.at[b], sem_x[b]).wait()

    def start_madd(ci, b):
        pltpu.async_copy(ntbl_hbm.at[idx_all.at[pl.ds(ci * _C, _C)]],
                         x_v.at[b], sem_m[b], add=True)

    def wait_madd(b):
        pltpu.make_async_copy(ntbl_hbm.at[pl.ds(0, _C)],
                              x_v.at[b], sem_m[b]).wait()

    def compute(b, accs):
        xb = x_v.at[b]

        def row_body(r2, a):
            new = list(a)
            for h in range(2):
                for j in range(_JREGS):
                    dv = xb[r2 * 2 + h, pl.ds(j * _L, _L)]
                    new[j] = new[j] + dv * dv
            return tuple(new)

        return lax.fori_loop(0, _C // 2, row_body, accs)

    for b in range(_NBUF):
        start_x(b, b)

    pltpu.sync_copy(lbl_hbm.at[pl.ds(base, _ROWS_PER_W)], idx_all)
    bias = cid * _NUM_CLASSES

    def bias_body(i, carry):
        iv = idx_all[pl.ds(i * _L, _L)]
        idx_all[pl.ds(i * _L, _L)] = iv + bias
        return carry

    lax.fori_loop(0, _ROWS_PER_W // _L, bias_body, 0)

    trow = sid * _TROWS
    nchunks = jnp.where(sid == _NS - 1, 5, _TROWS // _NEGC)

    def neg_chunk(k, carry):
        r0 = trow + k * _NEGC
        pltpu.sync_copy(tbl_hbm.at[pl.ds(r0, _NEGC)], neg_v)

        def neg_body(r, c2):
            for j in range(_JREGS):
                neg_v[r, pl.ds(j * _L, _L)] = -neg_v[r, pl.ds(j * _L, _L)]
            return c2

        lax.fori_loop(0, _NEGC, neg_body, 0)
        pltpu.sync_copy(neg_v, ntbl_hbm.at[pl.ds(bias + r0, _NEGC)])
        return carry

    lax.fori_loop(0, nchunks, neg_chunk, 0)
    plsc.subcore_barrier()

    for ci in range(_GLAG):
        wait_x(ci, ci)
        start_madd(ci, ci)

    def ring_turn(t, accs):
        for b in range(_NBUF):
            ci = t * _NBUF + b
            wait_madd(b)
            accs = compute(b, accs)
            start_x(ci + _NBUF, b)
            bg = (b + _GLAG) % _NBUF
            wait_x(ci + _GLAG, bg)
            start_madd(ci + _GLAG, bg)
        return accs

    zero = jnp.zeros((_L,), jnp.float32)
    accs = lax.fori_loop(0, _MAIN_T, ring_turn, (zero,) * _JREGS)

    for k in range(_NBUF):
        ci = _MAIN_T * _NBUF + k
        b = ci % _NBUF
        wait_madd(b)
        accs = compute(b, accs)
        if ci + _GLAG < _NCHUNK:
            bg = (ci + _GLAG) % _NBUF
            wait_x(ci + _GLAG, bg)
            start_madd(ci + _GLAG, bg)

    total = accs[0]
    for j in range(1, _JREGS):
        total = total + accs[j]
    acc_v[...] = total
    pltpu.sync_copy(acc_v, out_hbm.at[wid])


def kernel(inputs, labels, running_mean):
    partials, _ = _sc_sqdist(inputs, labels.astype(jnp.int32), running_mean)
    regularization = jnp.sqrt(jnp.sum(partials))
    return inputs, regularization


# speedup vs baseline: 1.2978x; 1.2609x over previous
import functools

import jax
import jax.numpy as jnp
from jax import lax
from jax.experimental import pallas as pl
from jax.experimental.pallas import tpu as pltpu
from jax.experimental.pallas import tpu_sc as plsc

_NUM_CLASSES = 10000
_D = 128
_N = 320000
_NC, _NS, _L = 2, 16, 16
_NW = _NC * _NS
_ROWS_PER_W = _N // _NW
_C = 40
_NCHUNK = _ROWS_PER_W // _C
_NBUF = 5
_MAIN_T = _NCHUNK // _NBUF - 1
_JREGS = _D // _L


@functools.partial(
    pl.kernel,
    out_type=jax.ShapeDtypeStruct((_NW, _L), jnp.float32),
    mesh=plsc.VectorSubcoreMesh(
        core_axis_name="c", subcore_axis_name="s",
        num_cores=_NC, num_subcores=_NS),
    scratch_types=[
        pltpu.VMEM((_ROWS_PER_W,), jnp.int32),
        pltpu.VMEM((_NBUF, _C, _D), jnp.float32),
        pltpu.VMEM((_NBUF, _C, _D), jnp.float32),
        pltpu.VMEM((_L,), jnp.float32),
    ] + [pltpu.SemaphoreType.DMA] * (2 * _NBUF),
)
def _sc_sqdist(x_hbm, lbl_hbm, tbl_hbm, out_hbm,
               idx_all, x_v, m_v, acc_v, *sems):
    sem_x = sems[:_NBUF]
    sem_m = sems[_NBUF:]
    wid = lax.axis_index("s") * _NC + lax.axis_index("c")
    base = wid * _ROWS_PER_W

    pltpu.sync_copy(lbl_hbm.at[pl.ds(base, _ROWS_PER_W)], idx_all)

    def start(ci, b):
        row0 = base + ci * _C
        pltpu.async_copy(x_hbm.at[pl.ds(row0, _C)], x_v.at[b], sem_x[b])
        pltpu.async_copy(tbl_hbm.at[idx_all.at[pl.ds(ci * _C, _C)]],
                         m_v.at[b], sem_m[b])

    def wait(ci, b):
        row0 = base + ci * _C
        pltpu.make_async_copy(x_hbm.at[pl.ds(row0, _C)],
                              x_v.at[b], sem_x[b]).wait()
        pltpu.make_async_copy(tbl_hbm.at[pl.ds(0, _C)],
                              m_v.at[b], sem_m[b]).wait()

    def compute(b, accs):
        xb = x_v.at[b]
        mb = m_v.at[b]

        def row_body(r, a):
            new = []
            for j in range(_JREGS):
                dv = xb[r, pl.ds(j * _L, _L)] - mb[r, pl.ds(j * _L, _L)]
                new.append(a[j] + dv * dv)
            return tuple(new)

        return lax.fori_loop(0, _C, row_body, accs)

    for b in range(_NBUF):
        start(b, b)

    def ring_turn(t, accs):
        for b in range(_NBUF):
            ci = t * _NBUF + b
            wait(ci, b)
            accs = compute(b, accs)
            start(ci + _NBUF, b)
        return accs

    zero = jnp.zeros((_L,), jnp.float32)
    accs = lax.fori_loop(0, _MAIN_T, ring_turn, (zero,) * _JREGS)

    for b in range(_NBUF):
        ci = _MAIN_T * _NBUF + b
        wait(ci, b)
        accs = compute(b, accs)

    total = accs[0]
    for j in range(1, _JREGS):
        total = total + accs[j]
    acc_v[...] = total
    pltpu.sync_copy(acc_v, out_hbm.at[wid])


def kernel(inputs, labels, running_mean):
    partials = _sc_sqdist(inputs, labels.astype(jnp.int32), running_mean)
    regularization = jnp.sqrt(jnp.sum(partials))
    return inputs, regularization
